# baseline (speedup 1.0000x reference)
import functools

import jax
import jax.numpy as jnp
from jax import lax
from jax.experimental import pallas as pl
from jax.experimental.pallas import tpu as pltpu
from jax.experimental.pallas import tpu_sc as plsc

B, C, H, T, K = 10, 128, 128, 1000, 9
N = B * T
E = 320000
_EPS = 1e-5
_PREC = jax.lax.Precision.HIGHEST

_NC, _NS, _L = 2, 16, 16
_NW = _NC * _NS
_CH = 80
_EPW = E // _NW
_NCHUNK = _EPW // _CH
_NROW = E // _CH
_GRP = _CH // _L


def _bn3_body(x_ref, g_ref, b_ref, o_ref):
    x = x_ref[...]
    mean = jnp.mean(x, axis=(0, 2), keepdims=True)
    var = jnp.mean((x - mean) ** 2, axis=(0, 2), keepdims=True)
    o_ref[...] = (x - mean) * jax.lax.rsqrt(var + _EPS) * g_ref[...][None, :, :] \
        + b_ref[...][None, :, :]


def _mm2_body(x_ref, wl_ref, wr_ref, bl_ref, br_ref, att_ref,
              xl_ref, xr_ref, ml_ref, mr_ref):
    i = pl.program_id(0)
    a = x_ref[...]
    xl = jnp.dot(a, wl_ref[...], preferred_element_type=jnp.float32,
                 precision=_PREC) + bl_ref[...]
    xr = jnp.dot(a, wr_ref[...], preferred_element_type=jnp.float32,
                 precision=_PREC) + br_ref[...]
    xl_ref[...] = xl
    xr_ref[...] = xr

    attv = att_ref[...]
    aab = jnp.abs(attv)
    p = jnp.sum(xl * attv, axis=1, keepdims=True)
    u = jnp.sum(jnp.abs(xl) * aab, axis=1, keepdims=True)
    q = jnp.sum(xr * attv, axis=1, keepdims=True)
    v = jnp.sum(jnp.abs(xr) * aab, axis=1, keepdims=True)
    mls = jnp.max(0.6 * p + 0.4 * u)
    mrs = jnp.max(0.6 * q + 0.4 * v)

    @pl.when(i == 0)
    def _():
        ml_ref[...] = jnp.full((1, H), -jnp.inf, jnp.float32)
        mr_ref[...] = jnp.full((1, H), -jnp.inf, jnp.float32)

    ml_ref[...] = jnp.maximum(ml_ref[...], mls)
    mr_ref[...] = jnp.maximum(mr_ref[...], mrs)


def _convT_body(x_ref, w_ref, b_ref, o_ref, *, relu):
    xb = x_ref[0]
    zp = jnp.concatenate([jnp.zeros((K // 2, C), jnp.float32), xb,
                          jnp.zeros((K // 2, C), jnp.float32)], axis=0)
    acc = b_ref[...] * jnp.ones((T, 1), jnp.float32)
    for k in range(K):
        acc = acc + jnp.dot(zp[k:k + T, :], w_ref[k],
                            preferred_element_type=jnp.float32, precision=_PREC)
    if relu:
        acc = jnp.maximum(acc, 0.0)
    o_ref[0] = acc


def _bn2_body(g0_ref, g1_ref, d0_ref, d1_ref, bias_ref, gm_ref, bt_ref, o_ref):
    d = d0_ref[...] + d1_ref[...]
    gat = (g0_ref[...] + g1_ref[...]) / (d + 1e-16) + bias_ref[...]
    mean = jnp.mean(gat, axis=0, keepdims=True)
    var = jnp.mean((gat - mean) ** 2, axis=0, keepdims=True)
    o_ref[...] = jnp.maximum(
        (gat - mean) * jax.lax.rsqrt(var + _EPS) * gm_ref[...] + bt_ref[...], 0.0)


def _bn3b_body(cv_ref, res_ref, g_ref, b_ref, o_ref):
    cv = cv_ref[...]
    mean = jnp.mean(cv, axis=(0, 1), keepdims=True)
    var = jnp.mean((cv - mean) ** 2, axis=(0, 1), keepdims=True)
    h = jnp.maximum((cv - mean) * jax.lax.rsqrt(var + _EPS) * g_ref[...]
                    + b_ref[...], 0.0)
    o_ref[...] = res_ref[...] + h


def _sc_edge_body(xl_hbm, xr_hbm, sd_hbm, att_hbm, ml_hbm, mr_hbm,
                  zg_hbm, gout_hbm, dout_hbm,
                  idx4, bufL2, bufR2, exb2, att_v, mlv, mrv, dbuf,
                  gacc_sh, dacc_sh, semI, semL, semR, semS, semE):
    c = lax.axis_index("c")
    s = lax.axis_index("s")
    wid = c * _NS + s

    d_chunk = 624
    tail = N - (_NS - 1) * d_chunk

    for i in range(640 // _L):
        dbuf[pl.ds(i * _L, _L)] = jnp.zeros((_L,), jnp.float32)

    @pl.when(s < _NS - 1)
    def _():
        pltpu.sync_copy(zg_hbm.at[pl.ds(0, d_chunk)],
                        gacc_sh.at[pl.ds(s * d_chunk, d_chunk)])
        pltpu.sync_copy(dbuf.at[pl.ds(0, d_chunk)],
                        dacc_sh.at[pl.ds(s * d_chunk, d_chunk)])

    @pl.when(s == _NS - 1)
    def _():
        pltpu.sync_copy(zg_hbm, gacc_sh.at[pl.ds((_NS - 1) * d_chunk, tail)])
        pltpu.sync_copy(dbuf, dacc_sh.at[pl.ds((_NS - 1) * d_chunk, tail)])

    pltpu.sync_copy(att_hbm, att_v)
    pltpu.sync_copy(ml_hbm.at[pl.ds(0, _L)], mlv)
    pltpu.sync_copy(mr_hbm.at[pl.ds(0, _L)], mrv)
    m16 = mlv[...] + mrv[...]
    atts = [att_v[pl.ds(k * _L, _L)] for k in range(C // _L)]

    plsc.subcore_barrier()

    lane = jax.lax.iota(jnp.int32, _L)

    def idx_fetch(j):
        pltpu.async_copy(sd_hbm.at[wid, j], idx4.at[j & 3], semI.at[j & 3])

    def idx_wait(j):
        pltpu.make_async_copy(sd_hbm.at[wid, j], idx4.at[j & 3],
                              semI.at[j & 3]).wait()

    def gather_issue(j):
        p = j & 1
        pltpu.async_copy(xl_hbm.at[idx4.at[j & 3, 0]], bufL2.at[p], semL.at[p])
        pltpu.async_copy(xr_hbm.at[idx4.at[j & 3, 1]], bufR2.at[p], semR.at[p])

    def gather_wait(j):
        p = j & 1
        pltpu.make_async_copy(xl_hbm.at[idx4.at[j & 3, 0]], bufL2.at[p],
                              semL.at[p]).wait()
        pltpu.make_async_copy(xr_hbm.at[idx4.at[j & 3, 1]], bufR2.at[p],
                              semR.at[p]).wait()

    def scatter_issue(j):
        p = j & 1
        pltpu.async_copy(bufL2.at[p], gacc_sh.at[idx4.at[j & 3, 1]],
                         semS.at[p], add=True)
        pltpu.async_copy(exb2.at[p], dacc_sh.at[idx4.at[j & 3, 1]],
                         semE.at[p], add=True)

    def scatter_wait(j):
        p = j & 1
        pltpu.make_async_copy(bufL2.at[p], gacc_sh.at[idx4.at[j & 3, 1]],
                              semS.at[p]).wait()
        pltpu.make_async_copy(exb2.at[p], dacc_sh.at[idx4.at[j & 3, 1]],
                              semE.at[p]).wait()

    def compute(j):
        p = j & 1

        def gbody(g, carry):
            def ebody(le, res):
                e = g * _L + le
                avs = []
                ms = []
                for k in range(C // _L):
                    a = bufL2[p, e, pl.ds(k * _L, _L)]
                    b = bufR2[p, e, pl.ds(k * _L, _L)]
                    avs.append(a)
                    m = a + b
                    m = jnp.maximum(m, 0.2 * m)
                    ms.append(m * atts[k])
                t01 = ms[0] + ms[1]
                t23 = ms[2] + ms[3]
                t45 = ms[4] + ms[5]
                t67 = ms[6] + ms[7]
                acc = (t01 + t23) + (t45 + t67)
                tot = jnp.sum(acc)
                exv = jnp.exp(jnp.full((_L,), tot, jnp.float32) - m16)
                for k in range(C // _L):
                    bufL2[p, e, pl.ds(k * _L, _L)] = avs[k] * exv
                return jnp.where(lane == le, exv, res)

            res = lax.fori_loop(0, _L, ebody, jnp.zeros((_L,), jnp.float32),
                                unroll=2)
            exb2[p, pl.ds(g * _L, _L)] = res
            return carry

        lax.fori_loop(0, _GRP, gbody, 0)

    idx_fetch(0)
    idx_wait(0)
    gather_issue(0)
    idx_fetch(1)

    def chunk_step(j, carry):
        gather_wait(j)
        compute(j)

        @pl.when(j >= 1)
        def _():
            scatter_wait(j - 1)

        @pl.when(j + 2 < _NCHUNK)
        def _():
            idx_fetch(j + 2)

        @pl.when(j + 1 < _NCHUNK)
        def _():
            idx_wait(j + 1)
            gather_issue(j + 1)

        scatter_issue(j)
        return carry

    lax.fori_loop(0, _NCHUNK, chunk_step, 0)
    scatter_wait(_NCHUNK - 1)

    plsc.subcore_barrier()

    @pl.when(s < _NS - 1)
    def _():
        pltpu.sync_copy(gacc_sh.at[pl.ds(s * d_chunk, d_chunk)],
                        gout_hbm.at[pl.ds(c * N + s * d_chunk, d_chunk)])
        pltpu.sync_copy(dacc_sh.at[pl.ds(s * d_chunk, d_chunk)],
                        dbuf.at[pl.ds(0, d_chunk)])
        pltpu.sync_copy(dbuf.at[pl.ds(0, d_chunk)],
                        dout_hbm.at[pl.ds(c * N + s * d_chunk, d_chunk)])

    @pl.when(s == _NS - 1)
    def _():
        pltpu.sync_copy(gacc_sh.at[pl.ds((_NS - 1) * d_chunk, tail)],
                        gout_hbm.at[pl.ds(c * N + (_NS - 1) * d_chunk, tail)])
        pltpu.sync_copy(dacc_sh.at[pl.ds((_NS - 1) * d_chunk, tail)], dbuf)
        pltpu.sync_copy(dbuf,
                        dout_hbm.at[pl.ds(c * N + (_NS - 1) * d_chunk, tail)])


def _sc_edge(xl, xr, sd, att, ml, mr, zg):
    mesh = plsc.VectorSubcoreMesh(core_axis_name="c", subcore_axis_name="s",
                                  num_cores=_NC, num_subcores=_NS)
    f32 = jnp.float32
    i32 = jnp.int32
    call = pl.kernel(
        _sc_edge_body,
        out_type=[jax.ShapeDtypeStruct((_NC * N, H), f32),
                  jax.ShapeDtypeStruct((_NC * N,), f32)],
        mesh=mesh,
        compiler_params=pltpu.CompilerParams(needs_layout_passes=False),
        scratch_types=[
            pltpu.VMEM((4, 2, _CH), i32),
            pltpu.VMEM((2, _CH, H), f32),
            pltpu.VMEM((2, _CH, H), f32),
            pltpu.VMEM((2, _CH), f32),
            pltpu.VMEM((C,), f32),
            pltpu.VMEM((_L,), f32),
            pltpu.VMEM((_L,), f32),
            pltpu.VMEM((640,), f32),
            pltpu.VMEM_SHARED((N, H), f32),
            pltpu.VMEM_SHARED((N,), f32),
            pltpu.SemaphoreType.DMA((4,)),
            pltpu.SemaphoreType.DMA((2,)),
            pltpu.SemaphoreType.DMA((2,)),
            pltpu.SemaphoreType.DMA((2,)),
            pltpu.SemaphoreType.DMA((2,)),
        ],
    )
    return call(xl, xr, sd, att, ml, mr, zg)


def kernel(x, edge_index, train, W_l, b_l, W_r, b_r, att, bias_gat,
           gamma0, beta0, gamma1, beta1, Wt, bt, Ws, bs):
    f32 = jnp.float32

    xn = pl.pallas_call(
        _bn3_body,
        out_shape=jax.ShapeDtypeStruct((B, C, T), f32),
    )(x, gamma0.reshape(C, 1), beta0.reshape(C, 1))

    x2 = xn.reshape(N, C)
    xnT = jnp.swapaxes(xn, 1, 2)

    xl, xr, ml, mr = pl.pallas_call(
        _mm2_body,
        grid=(B,),
        in_specs=[
            pl.BlockSpec((T, C), lambda i: (i, 0)),
            pl.BlockSpec((C, H), lambda i: (0, 0)),
            pl.BlockSpec((C, H), lambda i: (0, 0)),
            pl.BlockSpec((1, H), lambda i: (0, 0)),
            pl.BlockSpec((1, H), lambda i: (0, 0)),
            pl.BlockSpec((1, H), lambda i: (0, 0)),
        ],
        out_specs=[
            pl.BlockSpec((T, H), lambda i: (i, 0)),
            pl.BlockSpec((T, H), lambda i: (i, 0)),
            pl.BlockSpec((1, H), lambda i: (0, 0)),
            pl.BlockSpec((1, H), lambda i: (0, 0)),
        ],
        out_shape=[
            jax.ShapeDtypeStruct((N, H), f32),
            jax.ShapeDtypeStruct((N, H), f32),
            jax.ShapeDtypeStruct((1, H), f32),
            jax.ShapeDtypeStruct((1, H), f32),
        ],
    )(x2, W_l.T, W_r.T, b_l.reshape(1, H), b_r.reshape(1, H),
      att.reshape(1, H))

    conv_call = lambda body, inp, w, b: pl.pallas_call(
        body,
        grid=(B,),
        in_specs=[
            pl.BlockSpec((1, T, C), lambda i: (i, 0, 0)),
            pl.BlockSpec((K, C, H), lambda i: (0, 0, 0)),
            pl.BlockSpec((1, H), lambda i: (0, 0)),
        ],
        out_specs=pl.BlockSpec((1, T, H), lambda i: (i, 0, 0)),
        out_shape=jax.ShapeDtypeStruct((B, T, H), f32),
    )(inp, w, b)

    residT = conv_call(functools.partial(_convT_body, relu=True),
                       xnT, jnp.transpose(Ws, (2, 1, 0)), bs.reshape(1, H))

    src2 = edge_index[0].reshape(_NW, _NCHUNK, _CH)
    dst2 = edge_index[1].reshape(_NW, _NCHUNK, _CH)
    sd = jnp.stack([src2, dst2], axis=2)
    zg = jnp.zeros((640, H), f32)
    gout, dout = _sc_edge(xl, xr, sd, att,
                          ml.reshape(H), mr.reshape(H), zg)

    h2 = pl.pallas_call(
        _bn2_body,
        out_shape=jax.ShapeDtypeStruct((N, H), f32),
    )(gout[:N], gout[N:], dout[:N, None], dout[N:, None],
      bias_gat.reshape(1, H), gamma1.reshape(1, H), beta1.reshape(1, H))

    h3T = jnp.swapaxes(h2.reshape(B, H, T), 1, 2)

    convT = conv_call(functools.partial(_convT_body, relu=False),
                      h3T, jnp.transpose(Wt, (2, 1, 0)), bt.reshape(1, H))

    outT = pl.pallas_call(
        _bn3b_body,
        out_shape=jax.ShapeDtypeStruct((B, T, H), f32),
    )(convT, residT, gamma1.reshape(1, 1, H), beta1.reshape(1, 1, H))

    return jnp.swapaxes(outT, 1, 2)


# speedup vs baseline: 1.1557x; 1.1557x over previous
import functools

import jax
import jax.numpy as jnp
from jax import lax
from jax.experimental import pallas as pl
from jax.experimental.pallas import tpu as pltpu
from jax.experimental.pallas import tpu_sc as plsc

B, C, H, T, K = 10, 128, 128, 1000, 9
N = B * T
E = 320000
_EPS = 1e-5
_PREC = jax.lax.Precision.HIGHEST

_NC, _NS, _L = 2, 16, 16
_NW = _NC * _NS
_CH = 80
_EPW = E // _NW
_NCHUNK = _EPW // _CH
_NROW = E // _CH
_GRP = _CH // _L


def _bn3_body(x_ref, g_ref, b_ref, o_ref):
    x = x_ref[...]
    mean = jnp.mean(x, axis=(0, 2), keepdims=True)
    var = jnp.mean((x - mean) ** 2, axis=(0, 2), keepdims=True)
    o_ref[...] = (x - mean) * jax.lax.rsqrt(var + _EPS) * g_ref[...][None, :, :] \
        + b_ref[...][None, :, :]


def _mm2_body(x_ref, wl_ref, wr_ref, bl_ref, br_ref, att_ref,
              xl_ref, xr_ref, ml_ref, mr_ref):
    i = pl.program_id(0)
    a = x_ref[...]
    xl = jnp.dot(a, wl_ref[...], preferred_element_type=jnp.float32,
                 precision=_PREC) + bl_ref[...]
    xr = jnp.dot(a, wr_ref[...], preferred_element_type=jnp.float32,
                 precision=_PREC) + br_ref[...]
    xl_ref[...] = xl
    xr_ref[...] = xr

    attv = att_ref[...]
    aab = jnp.abs(attv)
    p = jnp.sum(xl * attv, axis=1, keepdims=True)
    u = jnp.sum(jnp.abs(xl) * aab, axis=1, keepdims=True)
    q = jnp.sum(xr * attv, axis=1, keepdims=True)
    v = jnp.sum(jnp.abs(xr) * aab, axis=1, keepdims=True)
    mls = jnp.max(0.6 * p + 0.4 * u)
    mrs = jnp.max(0.6 * q + 0.4 * v)

    @pl.when(i == 0)
    def _():
        ml_ref[...] = jnp.full((1, H), -jnp.inf, jnp.float32)
        mr_ref[...] = jnp.full((1, H), -jnp.inf, jnp.float32)

    ml_ref[...] = jnp.maximum(ml_ref[...], mls)
    mr_ref[...] = jnp.maximum(mr_ref[...], mrs)


def _convT_body(x_ref, w_ref, b_ref, o_ref, *, relu):
    xb = x_ref[0]
    zp = jnp.concatenate([jnp.zeros((K // 2, C), jnp.float32), xb,
                          jnp.zeros((K // 2, C), jnp.float32)], axis=0)
    acc = b_ref[...] * jnp.ones((T, 1), jnp.float32)
    for k in range(K):
        acc = acc + jnp.dot(zp[k:k + T, :], w_ref[k],
                            preferred_element_type=jnp.float32, precision=_PREC)
    if relu:
        acc = jnp.maximum(acc, 0.0)
    o_ref[0] = acc


def _bn2_body(g0_ref, g1_ref, d0_ref, d1_ref, bias_ref, gm_ref, bt_ref, o_ref):
    d = d0_ref[...] + d1_ref[...]
    gat = (g0_ref[...] + g1_ref[...]) / (d + 1e-16) + bias_ref[...]
    mean = jnp.mean(gat, axis=0, keepdims=True)
    var = jnp.mean((gat - mean) ** 2, axis=0, keepdims=True)
    o_ref[...] = jnp.maximum(
        (gat - mean) * jax.lax.rsqrt(var + _EPS) * gm_ref[...] + bt_ref[...], 0.0)


def _bn3b_body(cv_ref, res_ref, g_ref, b_ref, o_ref):
    cv = cv_ref[...]
    mean = jnp.mean(cv, axis=(0, 1), keepdims=True)
    var = jnp.mean((cv - mean) ** 2, axis=(0, 1), keepdims=True)
    h = jnp.maximum((cv - mean) * jax.lax.rsqrt(var + _EPS) * g_ref[...]
                    + b_ref[...], 0.0)
    o_ref[...] = res_ref[...] + h


def _sc_edge_body(xl_hbm, xr_hbm, sd_hbm, att_hbm, ml_hbm, mr_hbm,
                  zg_hbm, gout_hbm, dout_hbm,
                  idx4, bufL2, bufR2, exb2, att_v, mlv, mrv, dbuf,
                  gacc_sh, dacc_sh, semI, semL, semR, semS, semE):
    c = lax.axis_index("c")
    s = lax.axis_index("s")
    wid = c * _NS + s

    d_chunk = 624
    tail = N - (_NS - 1) * d_chunk

    for i in range(640 // _L):
        dbuf[pl.ds(i * _L, _L)] = jnp.zeros((_L,), jnp.float32)

    @pl.when(s < _NS - 1)
    def _():
        pltpu.sync_copy(zg_hbm.at[pl.ds(0, d_chunk)],
                        gacc_sh.at[pl.ds(s * d_chunk, d_chunk)])
        pltpu.sync_copy(dbuf.at[pl.ds(0, d_chunk)],
                        dacc_sh.at[pl.ds(s * d_chunk, d_chunk)])

    @pl.when(s == _NS - 1)
    def _():
        pltpu.sync_copy(zg_hbm, gacc_sh.at[pl.ds((_NS - 1) * d_chunk, tail)])
        pltpu.sync_copy(dbuf, dacc_sh.at[pl.ds((_NS - 1) * d_chunk, tail)])

    pltpu.sync_copy(att_hbm, att_v)
    pltpu.sync_copy(ml_hbm.at[pl.ds(0, _L)], mlv)
    pltpu.sync_copy(mr_hbm.at[pl.ds(0, _L)], mrv)
    m16 = mlv[...] + mrv[...]
    atts = [att_v[pl.ds(k * _L, _L)] for k in range(C // _L)]

    plsc.subcore_barrier()

    lane = jax.lax.iota(jnp.int32, _L)

    def idx_fetch(j):
        pltpu.async_copy(sd_hbm.at[wid, j], idx4.at[j & 3], semI.at[j & 3])

    def idx_wait(j):
        pltpu.make_async_copy(sd_hbm.at[wid, j], idx4.at[j & 3],
                              semI.at[j & 3]).wait()

    def gather_issue(j):
        p = j & 1
        pltpu.async_copy(xl_hbm.at[idx4.at[j & 3, 0]], bufL2.at[p], semL.at[p])
        pltpu.async_copy(xr_hbm.at[idx4.at[j & 3, 1]], bufR2.at[p], semR.at[p])

    def gather_wait(j):
        p = j & 1
        pltpu.make_async_copy(xl_hbm.at[idx4.at[j & 3, 0]], bufL2.at[p],
                              semL.at[p]).wait()
        pltpu.make_async_copy(xr_hbm.at[idx4.at[j & 3, 1]], bufR2.at[p],
                              semR.at[p]).wait()

    def scatter_issue(j):
        p = j & 1
        pltpu.async_copy(bufL2.at[p], gacc_sh.at[idx4.at[j & 3, 1]],
                         semS.at[p], add=True)
        pltpu.async_copy(exb2.at[p], dacc_sh.at[idx4.at[j & 3, 1]],
                         semE.at[p], add=True)

    def scatter_wait(j):
        p = j & 1
        pltpu.make_async_copy(bufL2.at[p], gacc_sh.at[idx4.at[j & 3, 1]],
                              semS.at[p]).wait()
        pltpu.make_async_copy(exb2.at[p], dacc_sh.at[idx4.at[j & 3, 1]],
                              semE.at[p]).wait()

    def compute(j):
        p = j & 1

        def gbody(g, carry):
            def ebody(le, res):
                e = g * _L + le
                avs = []
                ms = []
                for k in range(C // _L):
                    a = bufL2[p, e, pl.ds(k * _L, _L)]
                    b = bufR2[p, e, pl.ds(k * _L, _L)]
                    avs.append(a)
                    m = a + b
                    m = jnp.maximum(m, 0.2 * m)
                    ms.append(m * atts[k])
                t01 = ms[0] + ms[1]
                t23 = ms[2] + ms[3]
                t45 = ms[4] + ms[5]
                t67 = ms[6] + ms[7]
                acc = (t01 + t23) + (t45 + t67)
                tot = jnp.sum(acc)
                exv = jnp.exp(jnp.full((_L,), tot, jnp.float32) - m16)
                for k in range(C // _L):
                    bufL2[p, e, pl.ds(k * _L, _L)] = avs[k] * exv
                return jnp.where(lane == le, exv, res)

            res = lax.fori_loop(0, _L, ebody, jnp.zeros((_L,), jnp.float32),
                                unroll=4)
            exb2[p, pl.ds(g * _L, _L)] = res
            return carry

        lax.fori_loop(0, _GRP, gbody, 0)

    idx_fetch(0)
    idx_wait(0)
    gather_issue(0)
    idx_fetch(1)

    def chunk_step(j, carry):
        gather_wait(j)

        @pl.when(j >= 1)
        def _():
            scatter_wait(j - 1)

        @pl.when(j + 1 < _NCHUNK)
        def _():
            idx_wait(j + 1)
            gather_issue(j + 1)

        compute(j)

        @pl.when(j + 2 < _NCHUNK)
        def _():
            idx_fetch(j + 2)

        scatter_issue(j)
        return carry

    lax.fori_loop(0, _NCHUNK, chunk_step, 0)
    scatter_wait(_NCHUNK - 1)

    plsc.subcore_barrier()

    @pl.when(s < _NS - 1)
    def _():
        pltpu.sync_copy(gacc_sh.at[pl.ds(s * d_chunk, d_chunk)],
                        gout_hbm.at[pl.ds(c * N + s * d_chunk, d_chunk)])
        pltpu.sync_copy(dacc_sh.at[pl.ds(s * d_chunk, d_chunk)],
                        dbuf.at[pl.ds(0, d_chunk)])
        pltpu.sync_copy(dbuf.at[pl.ds(0, d_chunk)],
                        dout_hbm.at[pl.ds(c * N + s * d_chunk, d_chunk)])

    @pl.when(s == _NS - 1)
    def _():
        pltpu.sync_copy(gacc_sh.at[pl.ds((_NS - 1) * d_chunk, tail)],
                        gout_hbm.at[pl.ds(c * N + (_NS - 1) * d_chunk, tail)])
        pltpu.sync_copy(dacc_sh.at[pl.ds((_NS - 1) * d_chunk, tail)], dbuf)
        pltpu.sync_copy(dbuf,
                        dout_hbm.at[pl.ds(c * N + (_NS - 1) * d_chunk, tail)])


def _sc_edge(xl, xr, sd, att, ml, mr, zg):
    mesh = plsc.VectorSubcoreMesh(core_axis_name="c", subcore_axis_name="s",
                                  num_cores=_NC, num_subcores=_NS)
    f32 = jnp.float32
    i32 = jnp.int32
    call = pl.kernel(
        _sc_edge_body,
        out_type=[jax.ShapeDtypeStruct((_NC * N, H), f32),
                  jax.ShapeDtypeStruct((_NC * N,), f32)],
        mesh=mesh,
        compiler_params=pltpu.CompilerParams(needs_layout_passes=False),
        scratch_types=[
            pltpu.VMEM((4, 2, _CH), i32),
            pltpu.VMEM((2, _CH, H), f32),
            pltpu.VMEM((2, _CH, H), f32),
            pltpu.VMEM((2, _CH), f32),
            pltpu.VMEM((C,), f32),
            pltpu.VMEM((_L,), f32),
            pltpu.VMEM((_L,), f32),
            pltpu.VMEM((640,), f32),
            pltpu.VMEM_SHARED((N, H), f32),
            pltpu.VMEM_SHARED((N,), f32),
            pltpu.SemaphoreType.DMA((4,)),
            pltpu.SemaphoreType.DMA((2,)),
            pltpu.SemaphoreType.DMA((2,)),
            pltpu.SemaphoreType.DMA((2,)),
            pltpu.SemaphoreType.DMA((2,)),
        ],
    )
    return call(xl, xr, sd, att, ml, mr, zg)


def kernel(x, edge_index, train, W_l, b_l, W_r, b_r, att, bias_gat,
           gamma0, beta0, gamma1, beta1, Wt, bt, Ws, bs):
    f32 = jnp.float32

    xn = pl.pallas_call(
        _bn3_body,
        out_shape=jax.ShapeDtypeStruct((B, C, T), f32),
    )(x, gamma0.reshape(C, 1), beta0.reshape(C, 1))

    x2 = xn.reshape(N, C)
    xnT = jnp.swapaxes(xn, 1, 2)

    xl, xr, ml, mr = pl.pallas_call(
        _mm2_body,
        grid=(B,),
        in_specs=[
            pl.BlockSpec((T, C), lambda i: (i, 0)),
            pl.BlockSpec((C, H), lambda i: (0, 0)),
            pl.BlockSpec((C, H), lambda i: (0, 0)),
            pl.BlockSpec((1, H), lambda i: (0, 0)),
            pl.BlockSpec((1, H), lambda i: (0, 0)),
            pl.BlockSpec((1, H), lambda i: (0, 0)),
        ],
        out_specs=[
            pl.BlockSpec((T, H), lambda i: (i, 0)),
            pl.BlockSpec((T, H), lambda i: (i, 0)),
            pl.BlockSpec((1, H), lambda i: (0, 0)),
            pl.BlockSpec((1, H), lambda i: (0, 0)),
        ],
        out_shape=[
            jax.ShapeDtypeStruct((N, H), f32),
            jax.ShapeDtypeStruct((N, H), f32),
            jax.ShapeDtypeStruct((1, H), f32),
            jax.ShapeDtypeStruct((1, H), f32),
        ],
    )(x2, W_l.T, W_r.T, b_l.reshape(1, H), b_r.reshape(1, H),
      att.reshape(1, H))

    conv_call = lambda body, inp, w, b: pl.pallas_call(
        body,
        grid=(B,),
        in_specs=[
            pl.BlockSpec((1, T, C), lambda i: (i, 0, 0)),
            pl.BlockSpec((K, C, H), lambda i: (0, 0, 0)),
            pl.BlockSpec((1, H), lambda i: (0, 0)),
        ],
        out_specs=pl.BlockSpec((1, T, H), lambda i: (i, 0, 0)),
        out_shape=jax.ShapeDtypeStruct((B, T, H), f32),
    )(inp, w, b)

    residT = conv_call(functools.partial(_convT_body, relu=True),
                       xnT, jnp.transpose(Ws, (2, 1, 0)), bs.reshape(1, H))

    src2 = edge_index[0].reshape(_NW, _NCHUNK, _CH)
    dst2 = edge_index[1].reshape(_NW, _NCHUNK, _CH)
    sd = jnp.stack([src2, dst2], axis=2)
    zg = jnp.zeros((640, H), f32)
    gout, dout = _sc_edge(xl, xr, sd, att,
                          ml.reshape(H), mr.reshape(H), zg)

    h2 = pl.pallas_call(
        _bn2_body,
        out_shape=jax.ShapeDtypeStruct((N, H), f32),
    )(gout[:N], gout[N:], dout[:N, None], dout[N:, None],
      bias_gat.reshape(1, H), gamma1.reshape(1, H), beta1.reshape(1, H))

    h3T = jnp.swapaxes(h2.reshape(B, H, T), 1, 2)

    convT = conv_call(functools.partial(_convT_body, relu=False),
                      h3T, jnp.transpose(Wt, (2, 1, 0)), bt.reshape(1, H))

    outT = pl.pallas_call(
        _bn3b_body,
        out_shape=jax.ShapeDtypeStruct((B, T, H), f32),
    )(convT, residT, gamma1.reshape(1, 1, H), beta1.reshape(1, 1, H))

    return jnp.swapaxes(outT, 1, 2)


# speedup vs baseline: 1.7073x; 1.4773x over previous
import functools

import jax
import jax.numpy as jnp
from jax import lax
from jax.experimental import pallas as pl
from jax.experimental.pallas import tpu as pltpu
from jax.experimental.pallas import tpu_sc as plsc

B, C, H, T, K = 10, 128, 128, 1000, 9
N = B * T
E = 320000
_EPS = 1e-5
_PREC = jax.lax.Precision.HIGHEST

_NC, _NS, _L = 2, 16, 16
_NW = _NC * _NS
_CH = 80
_EPW = E // _NW
_NCHUNK = _EPW // _CH
_NROW = E // _CH
_GRP = _CH // _L


def _bn3_body(x_ref, g_ref, b_ref, o_ref):
    x = x_ref[...]
    mean = jnp.mean(x, axis=(0, 2), keepdims=True)
    var = jnp.mean((x - mean) ** 2, axis=(0, 2), keepdims=True)
    o_ref[...] = (x - mean) * jax.lax.rsqrt(var + _EPS) * g_ref[...][None, :, :] \
        + b_ref[...][None, :, :]


def _mm2_body(x_ref, wl_ref, wr_ref, bl_ref, br_ref, att_ref,
              xl_ref, xr_ref, ml_ref, mr_ref):
    i = pl.program_id(0)
    a = x_ref[...]
    xl = jnp.dot(a, wl_ref[...], preferred_element_type=jnp.float32,
                 precision=_PREC) + bl_ref[...]
    xr = jnp.dot(a, wr_ref[...], preferred_element_type=jnp.float32,
                 precision=_PREC) + br_ref[...]
    xl_ref[...] = xl
    xr_ref[...] = xr

    attv = att_ref[...]
    aab = jnp.abs(attv)
    p = jnp.sum(xl * attv, axis=1, keepdims=True)
    u = jnp.sum(jnp.abs(xl) * aab, axis=1, keepdims=True)
    q = jnp.sum(xr * attv, axis=1, keepdims=True)
    v = jnp.sum(jnp.abs(xr) * aab, axis=1, keepdims=True)
    mls = jnp.max(0.6 * p + 0.4 * u)
    mrs = jnp.max(0.6 * q + 0.4 * v)

    @pl.when(i == 0)
    def _():
        ml_ref[...] = jnp.full((1, H), -jnp.inf, jnp.float32)
        mr_ref[...] = jnp.full((1, H), -jnp.inf, jnp.float32)

    ml_ref[...] = jnp.maximum(ml_ref[...], mls)
    mr_ref[...] = jnp.maximum(mr_ref[...], mrs)


def _convT_body(x_ref, w_ref, b_ref, o_ref, *, relu):
    xb = x_ref[0]
    zp = jnp.concatenate([jnp.zeros((K // 2, C), jnp.float32), xb,
                          jnp.zeros((K // 2, C), jnp.float32)], axis=0)
    acc = b_ref[...] * jnp.ones((T, 1), jnp.float32)
    for k in range(K):
        acc = acc + jnp.dot(zp[k:k + T, :], w_ref[k],
                            preferred_element_type=jnp.float32, precision=_PREC)
    if relu:
        acc = jnp.maximum(acc, 0.0)
    o_ref[0] = acc


def _bn2_body(g0_ref, g1_ref, d0_ref, d1_ref, bias_ref, gm_ref, bt_ref, o_ref):
    d = d0_ref[...] + d1_ref[...]
    gat = (g0_ref[...] + g1_ref[...]) / (d + 1e-16) + bias_ref[...]
    mean = jnp.mean(gat, axis=0, keepdims=True)
    var = jnp.mean((gat - mean) ** 2, axis=0, keepdims=True)
    o_ref[...] = jnp.maximum(
        (gat - mean) * jax.lax.rsqrt(var + _EPS) * gm_ref[...] + bt_ref[...], 0.0)


def _bn3b_body(cv_ref, res_ref, g_ref, b_ref, o_ref):
    cv = cv_ref[...]
    mean = jnp.mean(cv, axis=(0, 1), keepdims=True)
    var = jnp.mean((cv - mean) ** 2, axis=(0, 1), keepdims=True)
    h = jnp.maximum((cv - mean) * jax.lax.rsqrt(var + _EPS) * g_ref[...]
                    + b_ref[...], 0.0)
    o_ref[...] = res_ref[...] + h


def _sc_edge_body(xl_hbm, xr_hbm, sd_hbm, att_hbm, ml_hbm, mr_hbm,
                  zg_hbm, gout_hbm, dout_hbm,
                  idx4, bufL2, bufR2, exb2, exf2, att_v, mlv, mrv, dbuf,
                  gacc_sh, dacc_sh, semI, semL, semR, semS, semE):
    c = lax.axis_index("c")
    s = lax.axis_index("s")
    wid = c * _NS + s

    d_chunk = 624
    tail = N - (_NS - 1) * d_chunk

    for i in range(640 // _L):
        dbuf[pl.ds(i * _L, _L)] = jnp.zeros((_L,), jnp.float32)

    @pl.when(s < _NS - 1)
    def _():
        pltpu.sync_copy(zg_hbm.at[pl.ds(0, d_chunk)],
                        gacc_sh.at[pl.ds(s * d_chunk, d_chunk)])
        pltpu.sync_copy(dbuf.at[pl.ds(0, d_chunk)],
                        dacc_sh.at[pl.ds(s * d_chunk, d_chunk)])

    @pl.when(s == _NS - 1)
    def _():
        pltpu.sync_copy(zg_hbm, gacc_sh.at[pl.ds((_NS - 1) * d_chunk, tail)])
        pltpu.sync_copy(dbuf, dacc_sh.at[pl.ds((_NS - 1) * d_chunk, tail)])

    pltpu.sync_copy(att_hbm, att_v)
    pltpu.sync_copy(ml_hbm.at[pl.ds(0, _L)], mlv)
    pltpu.sync_copy(mr_hbm.at[pl.ds(0, _L)], mrv)
    m16 = mlv[...] + mrv[...]
    atts = [att_v[pl.ds(k * _L, _L)] for k in range(C // _L)]

    plsc.subcore_barrier()

    lane = jax.lax.iota(jnp.int32, _L)

    def idx_fetch(j):
        pltpu.async_copy(sd_hbm.at[wid, j], idx4.at[j & 3], semI.at[j & 3])

    def idx_wait(j):
        pltpu.make_async_copy(sd_hbm.at[wid, j], idx4.at[j & 3],
                              semI.at[j & 3]).wait()

    def gather_issue(j):
        p = j & 1
        pltpu.async_copy(xl_hbm.at[idx4.at[j & 3, 0]], bufL2.at[p], semL.at[p])
        pltpu.async_copy(xr_hbm.at[idx4.at[j & 3, 1]], bufR2.at[p], semR.at[p])

    def gather_wait(j):
        p = j & 1
        pltpu.make_async_copy(xl_hbm.at[idx4.at[j & 3, 0]], bufL2.at[p],
                              semL.at[p]).wait()
        pltpu.make_async_copy(xr_hbm.at[idx4.at[j & 3, 1]], bufR2.at[p],
                              semR.at[p]).wait()

    def scatter_issue(j):
        p = j & 1
        pltpu.async_copy(bufR2.at[p], gacc_sh.at[idx4.at[j & 3, 1]],
                         semS.at[p], add=True)
        pltpu.async_copy(exb2.at[p], dacc_sh.at[idx4.at[j & 3, 1]],
                         semE.at[p], add=True)

    def scatter_wait(j):
        p = j & 1
        pltpu.make_async_copy(bufR2.at[p], gacc_sh.at[idx4.at[j & 3, 1]],
                              semS.at[p]).wait()
        pltpu.make_async_copy(exb2.at[p], dacc_sh.at[idx4.at[j & 3, 1]],
                              semE.at[p]).wait()

    def compute(j):
        p = j & 1

        def ebody(e):
            ms = []
            for k in range(C // _L):
                a = bufL2[p, e, pl.ds(k * _L, _L)]
                b = bufR2[p, e, pl.ds(k * _L, _L)]
                m = a + b
                m = jnp.maximum(m, 0.2 * m)
                ms.append(m * atts[k])
            acc = ((ms[0] + ms[1]) + (ms[2] + ms[3])) \
                + ((ms[4] + ms[5]) + (ms[6] + ms[7]))
            cfwd = jnp.cumsum(acc)
            crev = lax.rev(jnp.cumsum(lax.rev(acc, (0,))), (0,))
            exv = jnp.exp((cfwd + crev) - acc - m16)
            exf2[p, pl.ds(e * _L, _L)] = exv

        plsc.parallel_loop(0, _CH, unroll=4)(ebody)

        pvec = jnp.full((_L,), p, jnp.int32)
        for g in range(_GRP):
            dg = plsc.load_gather(exf2, [pvec, lane * (_L + 1) + g * _L * _L])
            exb2[p, pl.ds(g * _L, _L)] = dg

        def sbody(e):
            exv = exf2[p, pl.ds(e * _L, _L)]
            for k in range(C // _L):
                bufR2[p, e, pl.ds(k * _L, _L)] = \
                    bufL2[p, e, pl.ds(k * _L, _L)] * exv

        plsc.parallel_loop(0, _CH, unroll=4)(sbody)

    idx_fetch(0)
    idx_wait(0)
    gather_issue(0)
    idx_fetch(1)

    def chunk_step(j, carry):
        gather_wait(j)

        @pl.when(j >= 1)
        def _():
            scatter_wait(j - 1)

        @pl.when(j + 1 < _NCHUNK)
        def _():
            idx_wait(j + 1)
            gather_issue(j + 1)

        compute(j)

        @pl.when(j + 2 < _NCHUNK)
        def _():
            idx_fetch(j + 2)

        scatter_issue(j)
        return carry

    lax.fori_loop(0, _NCHUNK, chunk_step, 0)
    scatter_wait(_NCHUNK - 1)

    plsc.subcore_barrier()

    @pl.when(s < _NS - 1)
    def _():
        pltpu.sync_copy(gacc_sh.at[pl.ds(s * d_chunk, d_chunk)],
                        gout_hbm.at[pl.ds(c * N + s * d_chunk, d_chunk)])
        pltpu.sync_copy(dacc_sh.at[pl.ds(s * d_chunk, d_chunk)],
                        dbuf.at[pl.ds(0, d_chunk)])
        pltpu.sync_copy(dbuf.at[pl.ds(0, d_chunk)],
                        dout_hbm.at[pl.ds(c * N + s * d_chunk, d_chunk)])

    @pl.when(s == _NS - 1)
    def _():
        pltpu.sync_copy(gacc_sh.at[pl.ds((_NS - 1) * d_chunk, tail)],
                        gout_hbm.at[pl.ds(c * N + (_NS - 1) * d_chunk, tail)])
        pltpu.sync_copy(dacc_sh.at[pl.ds((_NS - 1) * d_chunk, tail)], dbuf)
        pltpu.sync_copy(dbuf,
                        dout_hbm.at[pl.ds(c * N + (_NS - 1) * d_chunk, tail)])


def _sc_edge(xl, xr, sd, att, ml, mr, zg):
    mesh = plsc.VectorSubcoreMesh(core_axis_name="c", subcore_axis_name="s",
                                  num_cores=_NC, num_subcores=_NS)
    f32 = jnp.float32
    i32 = jnp.int32
    call = pl.kernel(
        _sc_edge_body,
        out_type=[jax.ShapeDtypeStruct((_NC * N, H), f32),
                  jax.ShapeDtypeStruct((_NC * N,), f32)],
        mesh=mesh,
        compiler_params=pltpu.CompilerParams(needs_layout_passes=False),
        scratch_types=[
            pltpu.VMEM((4, 2, _CH), i32),
            pltpu.VMEM((2, _CH, H), f32),
            pltpu.VMEM((2, _CH, H), f32),
            pltpu.VMEM((2, _CH), f32),
            pltpu.VMEM((2, _CH * _L), f32),
            pltpu.VMEM((C,), f32),
            pltpu.VMEM((_L,), f32),
            pltpu.VMEM((_L,), f32),
            pltpu.VMEM((640,), f32),
            pltpu.VMEM_SHARED((N, H), f32),
            pltpu.VMEM_SHARED((N,), f32),
            pltpu.SemaphoreType.DMA((4,)),
            pltpu.SemaphoreType.DMA((2,)),
            pltpu.SemaphoreType.DMA((2,)),
            pltpu.SemaphoreType.DMA((2,)),
            pltpu.SemaphoreType.DMA((2,)),
        ],
    )
    return call(xl, xr, sd, att, ml, mr, zg)


def kernel(x, edge_index, train, W_l, b_l, W_r, b_r, att, bias_gat,
           gamma0, beta0, gamma1, beta1, Wt, bt, Ws, bs):
    f32 = jnp.float32

    xn = pl.pallas_call(
        _bn3_body,
        out_shape=jax.ShapeDtypeStruct((B, C, T), f32),
    )(x, gamma0.reshape(C, 1), beta0.reshape(C, 1))

    x2 = xn.reshape(N, C)
    xnT = jnp.swapaxes(xn, 1, 2)

    xl, xr, ml, mr = pl.pallas_call(
        _mm2_body,
        grid=(B,),
        in_specs=[
            pl.BlockSpec((T, C), lambda i: (i, 0)),
            pl.BlockSpec((C, H), lambda i: (0, 0)),
            pl.BlockSpec((C, H), lambda i: (0, 0)),
            pl.BlockSpec((1, H), lambda i: (0, 0)),
            pl.BlockSpec((1, H), lambda i: (0, 0)),
            pl.BlockSpec((1, H), lambda i: (0, 0)),
        ],
        out_specs=[
            pl.BlockSpec((T, H), lambda i: (i, 0)),
            pl.BlockSpec((T, H), lambda i: (i, 0)),
            pl.BlockSpec((1, H), lambda i: (0, 0)),
            pl.BlockSpec((1, H), lambda i: (0, 0)),
        ],
        out_shape=[
            jax.ShapeDtypeStruct((N, H), f32),
            jax.ShapeDtypeStruct((N, H), f32),
            jax.ShapeDtypeStruct((1, H), f32),
            jax.ShapeDtypeStruct((1, H), f32),
        ],
    )(x2, W_l.T, W_r.T, b_l.reshape(1, H), b_r.reshape(1, H),
      att.reshape(1, H))

    conv_call = lambda body, inp, w, b: pl.pallas_call(
        body,
        grid=(B,),
        in_specs=[
            pl.BlockSpec((1, T, C), lambda i: (i, 0, 0)),
            pl.BlockSpec((K, C, H), lambda i: (0, 0, 0)),
            pl.BlockSpec((1, H), lambda i: (0, 0)),
        ],
        out_specs=pl.BlockSpec((1, T, H), lambda i: (i, 0, 0)),
        out_shape=jax.ShapeDtypeStruct((B, T, H), f32),
    )(inp, w, b)

    residT = conv_call(functools.partial(_convT_body, relu=True),
                       xnT, jnp.transpose(Ws, (2, 1, 0)), bs.reshape(1, H))

    src2 = edge_index[0].reshape(_NW, _NCHUNK, _CH)
    dst2 = edge_index[1].reshape(_NW, _NCHUNK, _CH)
    sd = jnp.stack([src2, dst2], axis=2)
    zg = jnp.zeros((640, H), f32)
    gout, dout = _sc_edge(xl, xr, sd, att,
                          ml.reshape(H), mr.reshape(H), zg)

    h2 = pl.pallas_call(
        _bn2_body,
        out_shape=jax.ShapeDtypeStruct((N, H), f32),
    )(gout[:N], gout[N:], dout[:N, None], dout[N:, None],
      bias_gat.reshape(1, H), gamma1.reshape(1, H), beta1.reshape(1, H))

    h3T = jnp.swapaxes(h2.reshape(B, H, T), 1, 2)

    convT = conv_call(functools.partial(_convT_body, relu=False),
                      h3T, jnp.transpose(Wt, (2, 1, 0)), bt.reshape(1, H))

    outT = pl.pallas_call(
        _bn3b_body,
        out_shape=jax.ShapeDtypeStruct((B, T, H), f32),
    )(convT, residT, gamma1.reshape(1, 1, H), beta1.reshape(1, 1, H))

    return jnp.swapaxes(outT, 1, 2)


# speedup vs baseline: 1.9579x; 1.1468x over previous
import functools

import jax
import jax.numpy as jnp
from jax import lax
from jax.experimental import pallas as pl
from jax.experimental.pallas import tpu as pltpu
from jax.experimental.pallas import tpu_sc as plsc

B, C, H, T, K = 10, 128, 128, 1000, 9
N = B * T
E = 320000
_EPS = 1e-5
_PREC = jax.lax.Precision.HIGHEST

_NC, _NS, _L = 2, 16, 16
_NW = _NC * _NS
_CH = 80
_EPW = E // _NW
_NCHUNK = _EPW // _CH
_NROW = E // _CH
_GRP = _CH // _L


def _bn3_body(x_ref, g_ref, b_ref, o_ref):
    x = x_ref[...]
    mean = jnp.mean(x, axis=(0, 2), keepdims=True)
    var = jnp.mean((x - mean) ** 2, axis=(0, 2), keepdims=True)
    o_ref[...] = (x - mean) * jax.lax.rsqrt(var + _EPS) * g_ref[...][None, :, :] \
        + b_ref[...][None, :, :]


def _mm2_body(x_ref, wl_ref, wr_ref, bl_ref, br_ref, att_ref,
              xl_ref, xr_ref, ml_ref, mr_ref):
    i = pl.program_id(0)
    a = x_ref[...]
    xl = jnp.dot(a, wl_ref[...], preferred_element_type=jnp.float32,
                 precision=_PREC) + bl_ref[...]
    xr = jnp.dot(a, wr_ref[...], preferred_element_type=jnp.float32,
                 precision=_PREC) + br_ref[...]
    xl_ref[...] = xl
    xr_ref[...] = xr

    attv = att_ref[...]
    aab = jnp.abs(attv)
    p = jnp.sum(xl * attv, axis=1, keepdims=True)
    u = jnp.sum(jnp.abs(xl) * aab, axis=1, keepdims=True)
    q = jnp.sum(xr * attv, axis=1, keepdims=True)
    v = jnp.sum(jnp.abs(xr) * aab, axis=1, keepdims=True)
    mls = jnp.max(0.6 * p + 0.4 * u)
    mrs = jnp.max(0.6 * q + 0.4 * v)

    @pl.when(i == 0)
    def _():
        ml_ref[...] = jnp.full((1, H), -jnp.inf, jnp.float32)
        mr_ref[...] = jnp.full((1, H), -jnp.inf, jnp.float32)

    ml_ref[...] = jnp.maximum(ml_ref[...], mls)
    mr_ref[...] = jnp.maximum(mr_ref[...], mrs)


def _convT_body(x_ref, w_ref, b_ref, o_ref, *, relu):
    xb = x_ref[0]
    zp = jnp.concatenate([jnp.zeros((K // 2, C), jnp.float32), xb,
                          jnp.zeros((K // 2, C), jnp.float32)], axis=0)
    acc = b_ref[...] * jnp.ones((T, 1), jnp.float32)
    for k in range(K):
        acc = acc + jnp.dot(zp[k:k + T, :], w_ref[k],
                            preferred_element_type=jnp.float32,
                            precision=jax.lax.Precision.DEFAULT)
    if relu:
        acc = jnp.maximum(acc, 0.0)
    o_ref[0] = acc


def _bn2_body(g0_ref, g1_ref, d0_ref, d1_ref, bias_ref, gm_ref, bt_ref, o_ref):
    d = d0_ref[...] + d1_ref[...]
    gat = (g0_ref[...] + g1_ref[...]) / (d + 1e-16) + bias_ref[...]
    mean = jnp.mean(gat, axis=0, keepdims=True)
    var = jnp.mean((gat - mean) ** 2, axis=0, keepdims=True)
    o_ref[...] = jnp.maximum(
        (gat - mean) * jax.lax.rsqrt(var + _EPS) * gm_ref[...] + bt_ref[...], 0.0)


def _bn3b_body(cv_ref, res_ref, g_ref, b_ref, o_ref):
    cv = cv_ref[...]
    mean = jnp.mean(cv, axis=(0, 1), keepdims=True)
    var = jnp.mean((cv - mean) ** 2, axis=(0, 1), keepdims=True)
    h = jnp.maximum((cv - mean) * jax.lax.rsqrt(var + _EPS) * g_ref[...]
                    + b_ref[...], 0.0)
    o_ref[...] = res_ref[...] + h


def _sc_edge_body(xl_hbm, xr_hbm, sd_hbm, att_hbm, ml_hbm, mr_hbm,
                  zg_hbm, gout_hbm, dout_hbm,
                  idx4, bufL2, bufR2, exb2, exf2, att_v, mlv, mrv, dbuf,
                  gacc_sh, dacc_sh, semI, semL, semR, semS, semE):
    c = lax.axis_index("c")
    s = lax.axis_index("s")
    wid = c * _NS + s

    d_chunk = 624
    tail = N - (_NS - 1) * d_chunk

    for i in range(640 // _L):
        dbuf[pl.ds(i * _L, _L)] = jnp.zeros((_L,), jnp.float32)

    @pl.when(s < _NS - 1)
    def _():
        pltpu.sync_copy(zg_hbm.at[pl.ds(0, d_chunk)],
                        gacc_sh.at[pl.ds(s * d_chunk, d_chunk)])
        pltpu.sync_copy(dbuf.at[pl.ds(0, d_chunk)],
                        dacc_sh.at[pl.ds(s * d_chunk, d_chunk)])

    @pl.when(s == _NS - 1)
    def _():
        pltpu.sync_copy(zg_hbm, gacc_sh.at[pl.ds((_NS - 1) * d_chunk, tail)])
        pltpu.sync_copy(dbuf, dacc_sh.at[pl.ds((_NS - 1) * d_chunk, tail)])

    pltpu.sync_copy(att_hbm, att_v)
    pltpu.sync_copy(ml_hbm.at[pl.ds(0, _L)], mlv)
    pltpu.sync_copy(mr_hbm.at[pl.ds(0, _L)], mrv)
    m16 = mlv[...] + mrv[...]
    atts = [att_v[pl.ds(k * _L, _L)] for k in range(C // _L)]

    plsc.subcore_barrier()

    lane = jax.lax.iota(jnp.int32, _L)

    def idx_fetch(j):
        pltpu.async_copy(sd_hbm.at[wid, j], idx4.at[j & 3], semI.at[j & 3])

    def idx_wait(j):
        pltpu.make_async_copy(sd_hbm.at[wid, j], idx4.at[j & 3],
                              semI.at[j & 3]).wait()

    def gather_issue(j):
        p = j & 1
        pltpu.async_copy(xl_hbm.at[idx4.at[j & 3, 0]], bufL2.at[p], semL.at[p])
        pltpu.async_copy(xr_hbm.at[idx4.at[j & 3, 1]], bufR2.at[p], semR.at[p])

    def gather_wait(j):
        p = j & 1
        pltpu.make_async_copy(xl_hbm.at[idx4.at[j & 3, 0]], bufL2.at[p],
                              semL.at[p]).wait()
        pltpu.make_async_copy(xr_hbm.at[idx4.at[j & 3, 1]], bufR2.at[p],
                              semR.at[p]).wait()

    def scatter_issue(j):
        p = j & 1
        pltpu.async_copy(bufR2.at[p], gacc_sh.at[idx4.at[j & 3, 1]],
                         semS.at[p], add=True)
        pltpu.async_copy(exb2.at[p], dacc_sh.at[idx4.at[j & 3, 1]],
                         semE.at[p], add=True)

    def scatter_wait(j):
        p = j & 1
        pltpu.make_async_copy(bufR2.at[p], gacc_sh.at[idx4.at[j & 3, 1]],
                              semS.at[p]).wait()
        pltpu.make_async_copy(exb2.at[p], dacc_sh.at[idx4.at[j & 3, 1]],
                              semE.at[p]).wait()

    def compute(j):
        p = j & 1

        def ebody(e):
            ms = []
            for k in range(C // _L):
                a = bufL2[p, e, pl.ds(k * _L, _L)]
                b = bufR2[p, e, pl.ds(k * _L, _L)]
                m = a + b
                m = jnp.maximum(m, 0.2 * m)
                ms.append(m * atts[k])
            acc = ((ms[0] + ms[1]) + (ms[2] + ms[3])) \
                + ((ms[4] + ms[5]) + (ms[6] + ms[7]))
            cfwd = jnp.cumsum(acc)
            crev = lax.rev(jnp.cumsum(lax.rev(acc, (0,))), (0,))
            exv = jnp.exp((cfwd + crev) - acc - m16)
            exf2[p, pl.ds(e * _L, _L)] = exv

        plsc.parallel_loop(0, _CH, unroll=4)(ebody)

        pvec = jnp.full((_L,), p, jnp.int32)
        for g in range(_GRP):
            dg = plsc.load_gather(exf2, [pvec, lane * (_L + 1) + g * _L * _L])
            exb2[p, pl.ds(g * _L, _L)] = dg

        def sbody(e):
            exv = exf2[p, pl.ds(e * _L, _L)]
            for k in range(C // _L):
                bufR2[p, e, pl.ds(k * _L, _L)] = \
                    bufL2[p, e, pl.ds(k * _L, _L)] * exv

        plsc.parallel_loop(0, _CH, unroll=4)(sbody)

    idx_fetch(0)
    idx_wait(0)
    gather_issue(0)
    idx_fetch(1)

    def chunk_step(j, carry):
        gather_wait(j)

        @pl.when(j >= 1)
        def _():
            scatter_wait(j - 1)

        @pl.when(j + 1 < _NCHUNK)
        def _():
            idx_wait(j + 1)
            gather_issue(j + 1)

        compute(j)

        @pl.when(j + 2 < _NCHUNK)
        def _():
            idx_fetch(j + 2)

        scatter_issue(j)
        return carry

    lax.fori_loop(0, _NCHUNK, chunk_step, 0)
    scatter_wait(_NCHUNK - 1)

    plsc.subcore_barrier()

    @pl.when(s < _NS - 1)
    def _():
        pltpu.sync_copy(gacc_sh.at[pl.ds(s * d_chunk, d_chunk)],
                        gout_hbm.at[pl.ds(c * N + s * d_chunk, d_chunk)])
        pltpu.sync_copy(dacc_sh.at[pl.ds(s * d_chunk, d_chunk)],
                        dbuf.at[pl.ds(0, d_chunk)])
        pltpu.sync_copy(dbuf.at[pl.ds(0, d_chunk)],
                        dout_hbm.at[pl.ds(c * N + s * d_chunk, d_chunk)])

    @pl.when(s == _NS - 1)
    def _():
        pltpu.sync_copy(gacc_sh.at[pl.ds((_NS - 1) * d_chunk, tail)],
                        gout_hbm.at[pl.ds(c * N + (_NS - 1) * d_chunk, tail)])
        pltpu.sync_copy(dacc_sh.at[pl.ds((_NS - 1) * d_chunk, tail)], dbuf)
        pltpu.sync_copy(dbuf,
                        dout_hbm.at[pl.ds(c * N + (_NS - 1) * d_chunk, tail)])


def _sc_edge(xl, xr, sd, att, ml, mr, zg):
    mesh = plsc.VectorSubcoreMesh(core_axis_name="c", subcore_axis_name="s",
                                  num_cores=_NC, num_subcores=_NS)
    f32 = jnp.float32
    i32 = jnp.int32
    call = pl.kernel(
        _sc_edge_body,
        out_type=[jax.ShapeDtypeStruct((_NC * N, H), f32),
                  jax.ShapeDtypeStruct((_NC * N,), f32)],
        mesh=mesh,
        compiler_params=pltpu.CompilerParams(needs_layout_passes=False),
        scratch_types=[
            pltpu.VMEM((4, 2, _CH), i32),
            pltpu.VMEM((2, _CH, H), f32),
            pltpu.VMEM((2, _CH, H), f32),
            pltpu.VMEM((2, _CH), f32),
            pltpu.VMEM((2, _CH * _L), f32),
            pltpu.VMEM((C,), f32),
            pltpu.VMEM((_L,), f32),
            pltpu.VMEM((_L,), f32),
            pltpu.VMEM((640,), f32),
            pltpu.VMEM_SHARED((N, H), f32),
            pltpu.VMEM_SHARED((N,), f32),
            pltpu.SemaphoreType.DMA((4,)),
            pltpu.SemaphoreType.DMA((2,)),
            pltpu.SemaphoreType.DMA((2,)),
            pltpu.SemaphoreType.DMA((2,)),
            pltpu.SemaphoreType.DMA((2,)),
        ],
    )
    return call(xl, xr, sd, att, ml, mr, zg)


def kernel(x, edge_index, train, W_l, b_l, W_r, b_r, att, bias_gat,
           gamma0, beta0, gamma1, beta1, Wt, bt, Ws, bs):
    f32 = jnp.float32

    xn = pl.pallas_call(
        _bn3_body,
        out_shape=jax.ShapeDtypeStruct((B, C, T), f32),
    )(x, gamma0.reshape(C, 1), beta0.reshape(C, 1))

    x2 = xn.reshape(N, C)
    xnT = jnp.swapaxes(xn, 1, 2)

    xl, xr, ml, mr = pl.pallas_call(
        _mm2_body,
        grid=(B,),
        in_specs=[
            pl.BlockSpec((T, C), lambda i: (i, 0)),
            pl.BlockSpec((C, H), lambda i: (0, 0)),
            pl.BlockSpec((C, H), lambda i: (0, 0)),
            pl.BlockSpec((1, H), lambda i: (0, 0)),
            pl.BlockSpec((1, H), lambda i: (0, 0)),
            pl.BlockSpec((1, H), lambda i: (0, 0)),
        ],
        out_specs=[
            pl.BlockSpec((T, H), lambda i: (i, 0)),
            pl.BlockSpec((T, H), lambda i: (i, 0)),
            pl.BlockSpec((1, H), lambda i: (0, 0)),
            pl.BlockSpec((1, H), lambda i: (0, 0)),
        ],
        out_shape=[
            jax.ShapeDtypeStruct((N, H), f32),
            jax.ShapeDtypeStruct((N, H), f32),
            jax.ShapeDtypeStruct((1, H), f32),
            jax.ShapeDtypeStruct((1, H), f32),
        ],
    )(x2, W_l.T, W_r.T, b_l.reshape(1, H), b_r.reshape(1, H),
      att.reshape(1, H))

    conv_call = lambda body, inp, w, b: pl.pallas_call(
        body,
        grid=(B,),
        in_specs=[
            pl.BlockSpec((1, T, C), lambda i: (i, 0, 0)),
            pl.BlockSpec((K, C, H), lambda i: (0, 0, 0)),
            pl.BlockSpec((1, H), lambda i: (0, 0)),
        ],
        out_specs=pl.BlockSpec((1, T, H), lambda i: (i, 0, 0)),
        out_shape=jax.ShapeDtypeStruct((B, T, H), f32),
    )(inp, w, b)

    residT = conv_call(functools.partial(_convT_body, relu=True),
                       xnT, jnp.transpose(Ws, (2, 1, 0)), bs.reshape(1, H))

    src2 = edge_index[0].reshape(_NW, _NCHUNK, _CH)
    dst2 = edge_index[1].reshape(_NW, _NCHUNK, _CH)
    sd = jnp.stack([src2, dst2], axis=2)
    zg = jnp.zeros((640, H), f32)
    gout, dout = _sc_edge(xl, xr, sd, att,
                          ml.reshape(H), mr.reshape(H), zg)

    h2 = pl.pallas_call(
        _bn2_body,
        out_shape=jax.ShapeDtypeStruct((N, H), f32),
    )(gout[:N], gout[N:], dout[:N, None], dout[N:, None],
      bias_gat.reshape(1, H), gamma1.reshape(1, H), beta1.reshape(1, H))

    h3T = jnp.swapaxes(h2.reshape(B, H, T), 1, 2)

    convT = conv_call(functools.partial(_convT_body, relu=False),
                      h3T, jnp.transpose(Wt, (2, 1, 0)), bt.reshape(1, H))

    outT = pl.pallas_call(
        _bn3b_body,
        out_shape=jax.ShapeDtypeStruct((B, T, H), f32),
    )(convT, residT, gamma1.reshape(1, 1, H), beta1.reshape(1, 1, H))

    return jnp.swapaxes(outT, 1, 2)
